# baseline (speedup 1.0000x reference)
import functools

import jax
import jax.numpy as jnp
from jax import lax
from jax.experimental import pallas as pl
from jax.experimental.pallas import tpu as pltpu
from jax.experimental.pallas import tpu_sc as plsc


def _make_gather(B, V, D):
    info = plsc.get_sparse_core_info()
    NC, NS = info.num_cores, info.num_subcores
    NW = NC * NS
    assert B % NW == 0
    b_per_w = B // NW
    CHUNK = 128
    n_chunks = b_per_w // CHUNK

    mesh = plsc.VectorSubcoreMesh(core_axis_name="c", subcore_axis_name="s")

    @functools.partial(
        pl.kernel,
        mesh=mesh,
        compiler_params=pltpu.CompilerParams(use_tc_tiling_on_sc=False),
        out_type=jax.ShapeDtypeStruct((B, D), jnp.float32),
        scratch_types=[
            pltpu.VMEM((n_chunks, CHUNK), jnp.int32),
            pltpu.VMEM((2, CHUNK, D), jnp.float32),
            pltpu.SemaphoreType.DMA,
            pltpu.SemaphoreType.DMA,
        ],
    )
    def k(ids_hbm, table_hbm, out_hbm, idx_v, rows_v, sem0, sem1):
        sems = [sem0, sem1]
        wid = lax.axis_index("s") * NC + lax.axis_index("c")
        base = wid * b_per_w
        for j in range(n_chunks):
            pltpu.sync_copy(ids_hbm.at[pl.ds(base + j * CHUNK, CHUNK)], idx_v.at[j])
        copies = [None, None]
        copies[0] = pltpu.async_copy(table_hbm.at[idx_v.at[0]], rows_v.at[0], sems[0])
        for j in range(n_chunks):
            nxt = j + 1
            if nxt < n_chunks:
                copies[nxt % 2] = pltpu.async_copy(
                    table_hbm.at[idx_v.at[nxt]], rows_v.at[nxt % 2], sems[nxt % 2]
                )
            copies[j % 2].wait()
            pltpu.sync_copy(rows_v.at[j % 2], out_hbm.at[pl.ds(base + j * CHUNK, CHUNK)])

    return k


def kernel(style_ids, table):
    (B,) = style_ids.shape
    V, D = table.shape
    gather = _make_gather(B, V, D)
    return gather(style_ids.astype(jnp.int32), table)


# speedup vs baseline: 1.7338x; 1.7338x over previous
import functools

import jax
import jax.numpy as jnp
from jax import lax
from jax.experimental import pallas as pl
from jax.experimental.pallas import tpu as pltpu
from jax.experimental.pallas import tpu_sc as plsc


def _make_gather(B, V, D):
    info = plsc.get_sparse_core_info()
    NC, NS, L = info.num_cores, info.num_subcores, info.num_lanes
    NW = NC * NS
    assert B % NW == 0
    b_per_w = B // NW
    n_groups = b_per_w // L

    mesh = plsc.VectorSubcoreMesh(core_axis_name="c", subcore_axis_name="s")

    @functools.partial(
        pl.kernel,
        mesh=mesh,
        out_type=jax.ShapeDtypeStruct((B, D), jnp.float32),
        scratch_types=[
            pltpu.VMEM_SHARED((NW, b_per_w), jnp.int32),
            pltpu.SMEM((b_per_w,), jnp.int32),
            pltpu.VMEM((b_per_w, D), jnp.float32),
            pltpu.SemaphoreType.DMA,
        ],
    )
    def k(ids_hbm, table_hbm, out_hbm, idx_sh, idx_s, rows_v, sem):
        wid = lax.axis_index("s") * NC + lax.axis_index("c")
        base = wid * b_per_w
        pltpu.sync_copy(ids_hbm.at[pl.ds(base, b_per_w)], idx_sh.at[wid])
        pltpu.sync_copy(idx_sh.at[wid], idx_s)

        def body(j, carry):
            i = idx_s[j]
            pltpu.async_copy(
                table_hbm.at[pl.ds(i, 1)],
                rows_v.at[pl.ds(j, 1)],
                sem,
            )
            return carry

        lax.fori_loop(0, b_per_w, body, 0)
        pltpu.make_async_copy(
            table_hbm.at[pl.ds(0, b_per_w)], rows_v, sem
        ).wait()
        pltpu.sync_copy(rows_v, out_hbm.at[pl.ds(base, b_per_w)])

    return k


def kernel(style_ids, table):
    (B,) = style_ids.shape
    V, D = table.shape
    gather = _make_gather(B, V, D)
    return gather(style_ids.astype(jnp.int32), table)
